# baseline (speedup 1.0000x reference)
import math

import jax
import jax.numpy as jnp
from jax.experimental import pallas as pl
from jax.experimental.pallas import tpu as pltpu

_B, _Z, _D, _OUT = 64, 1024, 1024, 256


def _zoner_body(txt_ref, zone_ref, wt_ref, bt_ref, wz_ref, bz_ref,
                out_ref, t_ref):
    b = pl.program_id(0)

    @pl.when(b == 0)
    def _():
        t_ref[...] = jnp.tanh(
            jnp.dot(txt_ref[...], wt_ref[...],
                    preferred_element_type=jnp.float32) + bt_ref[...])

    z = jnp.tanh(
        jnp.dot(zone_ref[0], wz_ref[...],
                preferred_element_type=jnp.float32) + bz_ref[...])
    t_row = t_ref[pl.ds(b, 1), :]
    logits = jnp.sum(z * t_row, axis=1, keepdims=True)
    s = jnp.transpose(logits) * (1.0 / math.sqrt(_D))
    m = jnp.max(s, axis=1, keepdims=True)
    e = jnp.exp(s - m)
    out_ref[0] = e / jnp.sum(e, axis=1, keepdims=True)


def kernel(txt_embeds, zone_embeds, W_txt, b_txt, W_zone, b_zone):
    bt = b_txt.reshape(1, _OUT)
    bz = b_zone.reshape(1, _OUT)
    return pl.pallas_call(
        _zoner_body,
        grid=(_B,),
        in_specs=[
            pl.BlockSpec((_B, _D), lambda b: (0, 0)),
            pl.BlockSpec((1, _Z, _D), lambda b: (b, 0, 0)),
            pl.BlockSpec((_D, _OUT), lambda b: (0, 0)),
            pl.BlockSpec((1, _OUT), lambda b: (0, 0)),
            pl.BlockSpec((_D, _OUT), lambda b: (0, 0)),
            pl.BlockSpec((1, _OUT), lambda b: (0, 0)),
        ],
        out_specs=pl.BlockSpec((1, 1, _Z), lambda b: (b, 0, 0)),
        out_shape=jax.ShapeDtypeStruct((_B, 1, _Z), jnp.float32),
        scratch_shapes=[pltpu.VMEM((_B, _OUT), jnp.float32)],
    )(txt_embeds, zone_embeds, W_txt, bt, W_zone, bz).reshape(_B, _Z)
